# Initial kernel scaffold; baseline (speedup 1.0000x reference)
import jax
import jax.numpy as jnp
from jax.experimental import pallas as pl


def kernel(x, edge_index, Wp0, bp0, Ws0, Wn0, b0, Wp1, bp1, Ws1, Wn1, b1, Wp2, bp2, Ws2, Wn2, b2):
    raise NotImplementedError("write your pallas kernel here")


# speedup vs baseline: 1.0566x; 1.0566x over previous
import functools

import jax
import jax.numpy as jnp
from jax.experimental import pallas as pl
from jax.experimental.pallas import tpu as pltpu

N = 10000
D = 128


def _dense_pool_body(h_ref, w_ref, b_ref, o_ref):
    y = jax.lax.dot_general(
        h_ref[...], w_ref[...], (((1,), (0,)), ((), ())),
        preferred_element_type=jnp.float32)
    o_ref[...] = jnp.maximum(y + b_ref[...][None, :], 0.0)


def _dense_pool(h, w, b):
    return pl.pallas_call(
        _dense_pool_body,
        out_shape=jax.ShapeDtypeStruct((h.shape[0], w.shape[1]), jnp.float32),
    )(h, w, b)


def _dense_out_body(h_ref, agg_ref, ws_ref, wn_ref, b_ref, o_ref, *, act):
    y = jax.lax.dot_general(
        h_ref[...], ws_ref[...], (((1,), (0,)), ((), ())),
        preferred_element_type=jnp.float32)
    y = y + jax.lax.dot_general(
        agg_ref[...], wn_ref[...], (((1,), (0,)), ((), ())),
        preferred_element_type=jnp.float32)
    y = y + b_ref[...][None, :]
    if act == "relu":
        y = jnp.maximum(y, 0.0)
    elif act == "sigmoid":
        y = 1.0 / (1.0 + jnp.exp(-y))
    o_ref[...] = y


def _dense_out(h, agg, ws, wn, b, act):
    return pl.pallas_call(
        functools.partial(_dense_out_body, act=act),
        out_shape=jax.ShapeDtypeStruct((h.shape[0], ws.shape[1]), jnp.float32),
    )(h, agg, ws, wn, b)


def _segmax(hp, src, dst):
    msgs = hp[src]
    agg = jax.ops.segment_max(msgs, dst, num_segments=N)
    return jnp.where(jnp.isfinite(agg), agg, 0.0)


def kernel(x, edge_index, Wp0, bp0, Ws0, Wn0, b0, Wp1, bp1, Ws1, Wn1, b1,
           Wp2, bp2, Ws2, Wn2, b2):
    src = edge_index[0]
    dst = edge_index[1]
    Ws2p = jnp.pad(Ws2, ((0, 0), (0, D - Ws2.shape[1])))
    Wn2p = jnp.pad(Wn2, ((0, 0), (0, D - Wn2.shape[1])))
    b2p = jnp.pad(b2, (0, D - b2.shape[0]))
    params = [(Wp0, bp0, Ws0, Wn0, b0, "relu"),
              (Wp1, bp1, Ws1, Wn1, b1, "relu"),
              (Wp2, bp2, Ws2p, Wn2p, b2p, "sigmoid")]
    h = x
    for (Wp, bp, Ws, Wn, b, act) in params:
        hp = _dense_pool(h, Wp, bp)
        agg = _segmax(hp, src, dst)
        h = _dense_out(h, agg, Ws, Wn, b, act)
    return h[:, 0]


# speedup vs baseline: 2.0007x; 1.8936x over previous
import functools

import jax
import jax.numpy as jnp
from jax import lax
from jax.experimental import pallas as pl
from jax.experimental.pallas import tpu as pltpu
from jax.experimental.pallas import tpu_sc as plsc

N = 10000
D = 128
E = 320000
NC = 2
NS = 16
NW = NC * NS
RPW = 320
LASTR = N - (NW - 1) * RPW
CHUNK = 2560
NCHUNK = E // CHUNK
BATCH = 128
TRASH = RPW
CAP = E + NCHUNK * 16 + CHUNK + 16 + BATCH
CAP = ((CAP + 127) // 128) * 128

_mesh = plsc.VectorSubcoreMesh(core_axis_name="c", subcore_axis_name="s")


def _scalar(x):
    return jnp.max(x) if x.ndim else x


def _wid():
    return lax.axis_index("s") * NC + lax.axis_index("c")


def _bucket_body(src_hbm, dst_hbm, bsrc_hbm, bdl_hbm, bcnt_hbm,
                 src_v, dst_v, sel_src, sel_dl, pad_v, cnt_v):
    wid = _wid()
    lo = wid * RPW
    hi = jnp.minimum(lo + RPW, N)
    lanes = lax.iota(jnp.int32, 16)
    pad_src = wid * 16 + lanes
    trash_v = jnp.full((16,), TRASH, jnp.int32)

    def chunk_body(c, total):
        pltpu.sync_copy(src_hbm.at[pl.ds(c * CHUNK, CHUNK)], src_v)
        pltpu.sync_copy(dst_hbm.at[pl.ds(c * CHUNK, CHUNK)], dst_v)

        def grp(i, nsel):
            d = dst_v[pl.ds(i * 16, 16)]
            s = src_v[pl.ds(i * 16, 16)]
            m = (d >= lo) & (d < hi)
            c = plsc.cumsum(m.astype(jnp.int32))
            pos = nsel + c - 1
            plsc.store_scatter(sel_src, [pos], s, mask=m)
            plsc.store_scatter(sel_dl, [pos], d - lo, mask=m)
            return nsel + jnp.max(c)

        nsel = lax.fori_loop(0, CHUNK // 16, grp, 0)
        plsc.store_scatter(sel_src, [nsel + lanes], pad_src)
        plsc.store_scatter(sel_dl, [nsel + lanes], trash_v)
        nsel = jnp.bitwise_and(nsel + 15, -16)
        off = pl.multiple_of(wid * CAP + total, 16)
        pltpu.sync_copy(sel_src, bsrc_hbm.at[pl.ds(off, CHUNK + 16)])
        pltpu.sync_copy(sel_dl, bdl_hbm.at[pl.ds(off, CHUNK + 16)])
        return total + nsel

    total = lax.fori_loop(0, NCHUNK, chunk_body, 0)
    for k in range(BATCH // 16):
        pad_v[pl.ds(k * 16, 16)] = pad_src
    off = pl.multiple_of(wid * CAP + total, 16)
    pltpu.sync_copy(pad_v, bsrc_hbm.at[pl.ds(off, BATCH)])
    for k in range(BATCH // 16):
        pad_v[pl.ds(k * 16, 16)] = trash_v
    pltpu.sync_copy(pad_v, bdl_hbm.at[pl.ds(off, BATCH)])
    cnt_v[...] = jnp.full((16,), total, jnp.int32)
    pltpu.sync_copy(cnt_v, bcnt_hbm.at[pl.ds(pl.multiple_of(wid * 16, 16), 16)])


def _bucket(src, dst):
    f = pl.kernel(
        _bucket_body,
        out_type=(
            jax.ShapeDtypeStruct((NW * CAP,), jnp.int32),
            jax.ShapeDtypeStruct((NW * CAP,), jnp.int32),
            jax.ShapeDtypeStruct((NW * 16,), jnp.int32),
        ),
        mesh=_mesh,
        compiler_params=pltpu.CompilerParams(needs_layout_passes=False),
        scratch_types=[
            pltpu.VMEM((CHUNK,), jnp.int32),
            pltpu.VMEM((CHUNK,), jnp.int32),
            pltpu.VMEM((CHUNK + 16,), jnp.int32),
            pltpu.VMEM((CHUNK + 16,), jnp.int32),
            pltpu.VMEM((BATCH,), jnp.int32),
            pltpu.VMEM((16,), jnp.int32),
        ],
    )
    return f(src, dst)


def _segmax_body(hp_hbm, bsrc_hbm, bdl_hbm, bcnt_hbm, out_hbm,
                 agg_v, rows_v, idx_v, cnt_v, dl_v, shared, dl_smem, sem):
    wid = _wid()
    sid = lax.axis_index("s")
    lo = wid * RPW
    zero = jnp.zeros((16,), jnp.float32)

    def zr(r, _):
        for k in range(D // 16):
            agg_v[r, pl.ds(k * 16, 16)] = zero
        return 0

    lax.fori_loop(0, RPW + 1, zr, 0)

    pltpu.sync_copy(bcnt_hbm.at[pl.ds(pl.multiple_of(wid * 16, 16), 16)], cnt_v)
    cnt = jnp.max(cnt_v[...])
    nb = lax.div(cnt + BATCH - 1, BATCH)

    def batch(b, _):
        boff = pl.multiple_of(wid * CAP + b * BATCH, 16)
        pltpu.sync_copy(bsrc_hbm.at[pl.ds(boff, BATCH)], idx_v)
        pltpu.sync_copy(bdl_hbm.at[pl.ds(boff, BATCH)], dl_v)
        pltpu.sync_copy(dl_v, shared.at[sid])
        pltpu.sync_copy(shared.at[sid], dl_smem)
        pltpu.async_copy(hp_hbm.at[idx_v], rows_v, sem).wait()

        def upd(e, _):
            dd = dl_smem[e]
            for k in range(D // 16):
                sl = pl.ds(k * 16, 16)
                agg_v[dd, sl] = jnp.maximum(agg_v[dd, sl], rows_v[e, sl])
            return 0

        lax.fori_loop(0, BATCH, upd, 0)
        return 0

    lax.fori_loop(0, nb, batch, 0)

    @pl.when(wid < NW - 1)
    def _():
        pltpu.sync_copy(agg_v.at[pl.ds(0, RPW)], out_hbm.at[pl.ds(lo, RPW)])

    @pl.when(wid == NW - 1)
    def _():
        pltpu.sync_copy(agg_v.at[pl.ds(0, LASTR)], out_hbm.at[pl.ds(lo, LASTR)])


def _segmax(hp, bsrc, bdl, bcnt):
    f = pl.kernel(
        _segmax_body,
        out_type=jax.ShapeDtypeStruct((N, D), jnp.float32),
        mesh=_mesh,
        compiler_params=pltpu.CompilerParams(needs_layout_passes=False),
        scratch_types=[
            pltpu.VMEM((RPW + 1, D), jnp.float32),
            pltpu.VMEM((BATCH, D), jnp.float32),
            pltpu.VMEM((BATCH,), jnp.int32),
            pltpu.VMEM((16,), jnp.int32),
            pltpu.VMEM((BATCH,), jnp.int32),
            pltpu.VMEM_SHARED((NS, BATCH), jnp.int32),
            pltpu.SMEM((BATCH,), jnp.int32),
            pltpu.SemaphoreType.DMA,
        ],
    )
    return f(hp, bsrc, bdl, bcnt)


def _dense_pool_body(h_ref, w_ref, b_ref, o_ref):
    y = jax.lax.dot_general(
        h_ref[...], w_ref[...], (((1,), (0,)), ((), ())),
        preferred_element_type=jnp.float32)
    o_ref[...] = jnp.maximum(y + b_ref[...][None, :], 0.0)


def _dense_pool(h, w, b):
    return pl.pallas_call(
        _dense_pool_body,
        out_shape=jax.ShapeDtypeStruct((h.shape[0], w.shape[1]), jnp.float32),
    )(h, w, b)


def _dense_out_body(h_ref, agg_ref, ws_ref, wn_ref, b_ref, o_ref, *, act):
    y = jax.lax.dot_general(
        h_ref[...], ws_ref[...], (((1,), (0,)), ((), ())),
        preferred_element_type=jnp.float32)
    y = y + jax.lax.dot_general(
        agg_ref[...], wn_ref[...], (((1,), (0,)), ((), ())),
        preferred_element_type=jnp.float32)
    y = y + b_ref[...][None, :]
    if act == "relu":
        y = jnp.maximum(y, 0.0)
    elif act == "sigmoid":
        y = 1.0 / (1.0 + jnp.exp(-y))
    o_ref[...] = y


def _dense_out(h, agg, ws, wn, b, act):
    return pl.pallas_call(
        functools.partial(_dense_out_body, act=act),
        out_shape=jax.ShapeDtypeStruct((h.shape[0], ws.shape[1]), jnp.float32),
    )(h, agg, ws, wn, b)


def kernel(x, edge_index, Wp0, bp0, Ws0, Wn0, b0, Wp1, bp1, Ws1, Wn1, b1,
           Wp2, bp2, Ws2, Wn2, b2):
    src = edge_index[0]
    dst = edge_index[1]
    bsrc, bdl, bcnt = _bucket(src, dst)
    Ws2p = jnp.pad(Ws2, ((0, 0), (0, D - Ws2.shape[1])))
    Wn2p = jnp.pad(Wn2, ((0, 0), (0, D - Wn2.shape[1])))
    b2p = jnp.pad(b2, (0, D - b2.shape[0]))
    params = [(Wp0, bp0, Ws0, Wn0, b0, "relu"),
              (Wp1, bp1, Ws1, Wn1, b1, "relu"),
              (Wp2, bp2, Ws2p, Wn2p, b2p, "sigmoid")]
    h = x
    for (Wp, bp, Ws, Wn, b, act) in params:
        hp = _dense_pool(h, Wp, bp)
        agg = _segmax(hp, bsrc, bdl, bcnt)
        h = _dense_out(h, agg, Ws, Wn, b, act)
    return h[:, 0]


# speedup vs baseline: 2.2720x; 1.1356x over previous
import functools

import jax
import jax.numpy as jnp
from jax import lax
from jax.experimental import pallas as pl
from jax.experimental.pallas import tpu as pltpu
from jax.experimental.pallas import tpu_sc as plsc

N = 10000
D = 128
E = 320000
NC = 2
NS = 16
NW = NC * NS
RPW = 320
LASTR = N - (NW - 1) * RPW
CHUNK = 2560
NCHUNK = E // CHUNK
BATCH = 512
GUNROLL = 4
NSUB = BATCH // 128
TRASH = RPW
CAP = E + NCHUNK * 16 + CHUNK + 16 + BATCH
CAP = ((CAP + 127) // 128) * 128

_mesh = plsc.VectorSubcoreMesh(core_axis_name="c", subcore_axis_name="s")


def _scalar(x):
    return jnp.max(x) if x.ndim else x


def _wid():
    return lax.axis_index("s") * NC + lax.axis_index("c")


def _bucket_body(src_hbm, dst_hbm, bsrc_hbm, bdl_hbm, bcnt_hbm,
                 src_v, dst_v, sel_src, sel_dl, pad_v, cnt_v):
    wid = _wid()
    lo = wid * RPW
    hi = jnp.minimum(lo + RPW, N)
    lanes = lax.iota(jnp.int32, 16)
    pad_src = wid * 16 + lanes
    trash_v = jnp.full((16,), TRASH, jnp.int32)

    def chunk_body(c, total):
        pltpu.sync_copy(src_hbm.at[pl.ds(c * CHUNK, CHUNK)], src_v)
        pltpu.sync_copy(dst_hbm.at[pl.ds(c * CHUNK, CHUNK)], dst_v)

        def grp(i, nsel):
            d = dst_v[pl.ds(i * 16, 16)]
            s = src_v[pl.ds(i * 16, 16)]
            m = (d >= lo) & (d < hi)
            cs = plsc.cumsum(m.astype(jnp.int32))
            pos = nsel + cs - 1
            plsc.store_scatter(sel_src, [pos], s, mask=m)
            plsc.store_scatter(sel_dl, [pos], d - lo, mask=m)
            return nsel + jnp.max(cs)

        nsel = lax.fori_loop(0, CHUNK // 16, grp, 0)
        plsc.store_scatter(sel_src, [nsel + lanes], pad_src)
        plsc.store_scatter(sel_dl, [nsel + lanes], trash_v)
        nsel = jnp.bitwise_and(nsel + 15, -16)
        off = pl.multiple_of(wid * CAP + total, 16)
        pltpu.sync_copy(sel_src, bsrc_hbm.at[pl.ds(off, CHUNK + 16)])
        pltpu.sync_copy(sel_dl, bdl_hbm.at[pl.ds(off, CHUNK + 16)])
        return total + nsel

    total = lax.fori_loop(0, NCHUNK, chunk_body, 0)
    for k in range(BATCH // 16):
        pad_v[pl.ds(k * 16, 16)] = pad_src
    off = pl.multiple_of(wid * CAP + total, 16)
    pltpu.sync_copy(pad_v, bsrc_hbm.at[pl.ds(off, BATCH)])
    for k in range(BATCH // 16):
        pad_v[pl.ds(k * 16, 16)] = trash_v
    pltpu.sync_copy(pad_v, bdl_hbm.at[pl.ds(off, BATCH)])
    cnt_v[...] = jnp.full((16,), total, jnp.int32)
    pltpu.sync_copy(cnt_v, bcnt_hbm.at[pl.ds(pl.multiple_of(wid * 16, 16), 16)])


def _bucket(src, dst):
    f = pl.kernel(
        _bucket_body,
        out_type=(
            jax.ShapeDtypeStruct((NW * CAP,), jnp.int32),
            jax.ShapeDtypeStruct((NW * CAP,), jnp.int32),
            jax.ShapeDtypeStruct((NW * 16,), jnp.int32),
        ),
        mesh=_mesh,
        compiler_params=pltpu.CompilerParams(needs_layout_passes=False),
        scratch_types=[
            pltpu.VMEM((CHUNK,), jnp.int32),
            pltpu.VMEM((CHUNK,), jnp.int32),
            pltpu.VMEM((CHUNK + 16,), jnp.int32),
            pltpu.VMEM((CHUNK + 16,), jnp.int32),
            pltpu.VMEM((BATCH,), jnp.int32),
            pltpu.VMEM((16,), jnp.int32),
        ],
    )
    return f(src, dst)


def _segmax_body(hp_hbm, bsrc_hbm, bdl_hbm, bcnt_hbm, out_hbm,
                 agg_v, rows_v, idx_v, cnt_v, dl_v, shared, dl_smem, sem):
    wid = _wid()
    sid = lax.axis_index("s")
    lo = wid * RPW
    zero = jnp.zeros((16,), jnp.float32)

    def zr(r, _):
        for k in range(D // 16):
            agg_v[r, pl.ds(k * 16, 16)] = zero
        return 0

    lax.fori_loop(0, RPW + 1, zr, 0)

    pltpu.sync_copy(bcnt_hbm.at[pl.ds(pl.multiple_of(wid * 16, 16), 16)], cnt_v)
    cnt = jnp.max(cnt_v[...])
    nb = lax.div(cnt + BATCH - 1, BATCH)

    def batch(b, _):
        boff = pl.multiple_of(wid * CAP + b * BATCH, 16)
        for j in range(NSUB):
            pltpu.sync_copy(bsrc_hbm.at[pl.ds(boff + j * 128, 128)],
                            idx_v.at[j])
        pltpu.sync_copy(bdl_hbm.at[pl.ds(boff, BATCH)], dl_v)

        def issue(j):
            return pltpu.async_copy(hp_hbm.at[idx_v.at[j]], rows_v.at[j], sem)

        def update(j):
            pltpu.sync_copy(dl_v.at[pl.ds(j * 128, 128)], shared.at[sid])
            pltpu.sync_copy(shared.at[sid], dl_smem)

            def upd(e4, _):
                for u in range(4):
                    e = e4 * 4 + u
                    dd = dl_smem[e]
                    for k in range(D // 16):
                        sl = pl.ds(k * 16, 16)
                        agg_v[dd, sl] = jnp.maximum(agg_v[dd, sl],
                                                    rows_v[j, e, sl])
                return 0
            lax.fori_loop(0, 128 // 4, upd, 0)

        cp = issue(0)
        for j in range(NSUB):
            cp.wait()
            if j + 1 < NSUB:
                cp = issue(j + 1)
            update(j)
        return 0

    lax.fori_loop(0, nb, batch, 0)

    @pl.when(wid < NW - 1)
    def _():
        pltpu.sync_copy(agg_v.at[pl.ds(0, RPW)], out_hbm.at[pl.ds(lo, RPW)])

    @pl.when(wid == NW - 1)
    def _():
        pltpu.sync_copy(agg_v.at[pl.ds(0, LASTR)], out_hbm.at[pl.ds(lo, LASTR)])


def _segmax(hp, bsrc, bdl, bcnt):
    f = pl.kernel(
        _segmax_body,
        out_type=jax.ShapeDtypeStruct((N, D), jnp.float32),
        mesh=_mesh,
        compiler_params=pltpu.CompilerParams(needs_layout_passes=False),
        scratch_types=[
            pltpu.VMEM((RPW + 1, D), jnp.float32),
            pltpu.VMEM((NSUB, 128, D), jnp.float32),
            pltpu.VMEM((NSUB, 128), jnp.int32),
            pltpu.VMEM((16,), jnp.int32),
            pltpu.VMEM((BATCH,), jnp.int32),
            pltpu.VMEM_SHARED((NS, 128), jnp.int32),
            pltpu.SMEM((128,), jnp.int32),
            pltpu.SemaphoreType.DMA,
        ],
    )
    return f(hp, bsrc, bdl, bcnt)


def _dense_pool_body(h_ref, w_ref, b_ref, o_ref):
    y = jax.lax.dot_general(
        h_ref[...], w_ref[...], (((1,), (0,)), ((), ())),
        preferred_element_type=jnp.float32)
    o_ref[...] = jnp.maximum(y + b_ref[...][None, :], 0.0)


def _dense_pool(h, w, b):
    return pl.pallas_call(
        _dense_pool_body,
        out_shape=jax.ShapeDtypeStruct((h.shape[0], w.shape[1]), jnp.float32),
    )(h, w, b)


def _dense_out_body(h_ref, agg_ref, ws_ref, wn_ref, b_ref, o_ref, *, act):
    y = jax.lax.dot_general(
        h_ref[...], ws_ref[...], (((1,), (0,)), ((), ())),
        preferred_element_type=jnp.float32)
    y = y + jax.lax.dot_general(
        agg_ref[...], wn_ref[...], (((1,), (0,)), ((), ())),
        preferred_element_type=jnp.float32)
    y = y + b_ref[...][None, :]
    if act == "relu":
        y = jnp.maximum(y, 0.0)
    elif act == "sigmoid":
        y = 1.0 / (1.0 + jnp.exp(-y))
    o_ref[...] = y


def _dense_out(h, agg, ws, wn, b, act):
    return pl.pallas_call(
        functools.partial(_dense_out_body, act=act),
        out_shape=jax.ShapeDtypeStruct((h.shape[0], ws.shape[1]), jnp.float32),
    )(h, agg, ws, wn, b)


def kernel(x, edge_index, Wp0, bp0, Ws0, Wn0, b0, Wp1, bp1, Ws1, Wn1, b1,
           Wp2, bp2, Ws2, Wn2, b2):
    src = edge_index[0]
    dst = edge_index[1]
    bsrc, bdl, bcnt = _bucket(src, dst)
    Ws2p = jnp.pad(Ws2, ((0, 0), (0, D - Ws2.shape[1])))
    Wn2p = jnp.pad(Wn2, ((0, 0), (0, D - Wn2.shape[1])))
    b2p = jnp.pad(b2, (0, D - b2.shape[0]))
    params = [(Wp0, bp0, Ws0, Wn0, b0, "relu"),
              (Wp1, bp1, Ws1, Wn1, b1, "relu"),
              (Wp2, bp2, Ws2p, Wn2p, b2p, "sigmoid")]
    h = x
    for (Wp, bp, Ws, Wn, b, act) in params:
        hp = _dense_pool(h, Wp, bp)
        agg = _segmax(hp, bsrc, bdl, bcnt)
        h = _dense_out(h, agg, Ws, Wn, b, act)
    return h[:, 0]


# speedup vs baseline: 3.5617x; 1.5677x over previous
import functools

import jax
import jax.numpy as jnp
from jax import lax
from jax.experimental import pallas as pl
from jax.experimental.pallas import tpu as pltpu
from jax.experimental.pallas import tpu_sc as plsc

N = 10000
D = 128
E = 320000
NC = 2
NS = 16
NW = NC * NS
RPW = 320
LASTR = N - (NW - 1) * RPW
CHUNK = 2560
NCHUNK = E // CHUNK
BATCH = 512
GUNROLL = 4
NSUB = BATCH // 128
TRASH = RPW
CAP = E + NCHUNK * 16 + CHUNK + 16 + BATCH
CAP = ((CAP + 127) // 128) * 128

_mesh = plsc.VectorSubcoreMesh(core_axis_name="c", subcore_axis_name="s")


def _scalar(x):
    return jnp.max(x) if x.ndim else x


def _wid():
    return lax.axis_index("s") * NC + lax.axis_index("c")


def _bucket_body(src_hbm, dst_hbm, bsrc_hbm, bdl_hbm, bcnt_hbm,
                 src_v, dst_v, sel_src, sel_dl, pad_v, cnt_v):
    wid = _wid()
    lo = wid * RPW
    hi = jnp.minimum(lo + RPW, N)
    lanes = lax.iota(jnp.int32, 16)
    pad_src = wid * 16 + lanes
    trash_v = jnp.full((16,), TRASH, jnp.int32)

    def chunk_body(c, total):
        pltpu.sync_copy(src_hbm.at[pl.ds(c * CHUNK, CHUNK)], src_v)
        pltpu.sync_copy(dst_hbm.at[pl.ds(c * CHUNK, CHUNK)], dst_v)

        def grp(i, nsel_vec):
            for u in range(GUNROLL):
                g = i * GUNROLL + u
                d = dst_v[pl.ds(g * 16, 16)]
                s = src_v[pl.ds(g * 16, 16)]
                m = (d >= lo) & (d < hi)
                cs = plsc.cumsum(m.astype(jnp.int32))
                pos = nsel_vec + cs - 1
                plsc.store_scatter(sel_src, [pos], s, mask=m)
                plsc.store_scatter(sel_dl, [pos], d - lo, mask=m)
                pc = plsc.all_reduce_population_count(m)
                if pc.ndim == 0:
                    pc = jnp.full((16,), pc, jnp.int32)
                nsel_vec = nsel_vec + pc
            return nsel_vec

        nsel_vec = lax.fori_loop(0, CHUNK // (16 * GUNROLL), grp,
                                 jnp.zeros((16,), jnp.int32))
        plsc.store_scatter(sel_src, [nsel_vec + lanes], pad_src)
        plsc.store_scatter(sel_dl, [nsel_vec + lanes], trash_v)
        nsel = jnp.bitwise_and(jnp.max(nsel_vec) + 15, -16)
        off = pl.multiple_of(wid * CAP + total, 16)
        pltpu.sync_copy(sel_src, bsrc_hbm.at[pl.ds(off, CHUNK + 16)])
        pltpu.sync_copy(sel_dl, bdl_hbm.at[pl.ds(off, CHUNK + 16)])
        return total + nsel

    total = lax.fori_loop(0, NCHUNK, chunk_body, 0)
    for k in range(BATCH // 16):
        pad_v[pl.ds(k * 16, 16)] = pad_src
    off = pl.multiple_of(wid * CAP + total, 16)
    pltpu.sync_copy(pad_v, bsrc_hbm.at[pl.ds(off, BATCH)])
    for k in range(BATCH // 16):
        pad_v[pl.ds(k * 16, 16)] = trash_v
    pltpu.sync_copy(pad_v, bdl_hbm.at[pl.ds(off, BATCH)])
    cnt_v[...] = jnp.full((16,), total, jnp.int32)
    pltpu.sync_copy(cnt_v, bcnt_hbm.at[pl.ds(pl.multiple_of(wid * 16, 16), 16)])


def _bucket(src, dst):
    f = pl.kernel(
        _bucket_body,
        out_type=(
            jax.ShapeDtypeStruct((NW * CAP,), jnp.int32),
            jax.ShapeDtypeStruct((NW * CAP,), jnp.int32),
            jax.ShapeDtypeStruct((NW * 16,), jnp.int32),
        ),
        mesh=_mesh,
        compiler_params=pltpu.CompilerParams(needs_layout_passes=False),
        scratch_types=[
            pltpu.VMEM((CHUNK,), jnp.int32),
            pltpu.VMEM((CHUNK,), jnp.int32),
            pltpu.VMEM((CHUNK + 16,), jnp.int32),
            pltpu.VMEM((CHUNK + 16,), jnp.int32),
            pltpu.VMEM((BATCH,), jnp.int32),
            pltpu.VMEM((16,), jnp.int32),
        ],
    )
    return f(src, dst)


def _segmax_body(hp_hbm, bsrc_hbm, bdl_hbm, bcnt_hbm, out_hbm,
                 agg_v, rows_v, idx_v, cnt_v, dl_v, shared, dl_smem, sem):
    wid = _wid()
    sid = lax.axis_index("s")
    lo = wid * RPW
    zero = jnp.zeros((16,), jnp.float32)

    def zr(r, _):
        for k in range(D // 16):
            agg_v[r, pl.ds(k * 16, 16)] = zero
        return 0

    lax.fori_loop(0, RPW + 1, zr, 0)

    pltpu.sync_copy(bcnt_hbm.at[pl.ds(pl.multiple_of(wid * 16, 16), 16)], cnt_v)
    cnt = jnp.max(cnt_v[...])
    nb = lax.div(cnt + BATCH - 1, BATCH)

    def batch(b, _):
        boff = pl.multiple_of(wid * CAP + b * BATCH, 16)
        for j in range(NSUB):
            pltpu.sync_copy(bsrc_hbm.at[pl.ds(boff + j * 128, 128)],
                            idx_v.at[j])
        pltpu.sync_copy(bdl_hbm.at[pl.ds(boff, BATCH)], dl_v)

        def issue(j):
            return pltpu.async_copy(hp_hbm.at[idx_v.at[j]], rows_v.at[j], sem)

        def update(j):
            def upd(e4, _):
                for u in range(4):
                    e = e4 * 4 + u
                    dd = dl_smem[e]
                    sls = [pl.ds(k * 16, 16) for k in range(D // 16)]
                    a = [agg_v[dd, sl] for sl in sls]
                    r = [rows_v[j, e, sl] for sl in sls]
                    mx = [jnp.maximum(x, y) for x, y in zip(a, r)]
                    for sl, v in zip(sls, mx):
                        agg_v[dd, sl] = v
                return 0
            lax.fori_loop(0, 128 // 4, upd, 0)

        cp = issue(0)
        for j in range(NSUB):
            pltpu.sync_copy(dl_v.at[pl.ds(j * 128, 128)], shared.at[sid])
            pltpu.sync_copy(shared.at[sid], dl_smem)
            cp.wait()
            if j + 1 < NSUB:
                cp = issue(j + 1)
            update(j)
        return 0

    lax.fori_loop(0, nb, batch, 0)

    @pl.when(wid < NW - 1)
    def _():
        pltpu.sync_copy(agg_v.at[pl.ds(0, RPW)], out_hbm.at[pl.ds(lo, RPW)])

    @pl.when(wid == NW - 1)
    def _():
        pltpu.sync_copy(agg_v.at[pl.ds(0, LASTR)], out_hbm.at[pl.ds(lo, LASTR)])


def _segmax(hp, bsrc, bdl, bcnt):
    f = pl.kernel(
        _segmax_body,
        out_type=jax.ShapeDtypeStruct((N, D), jnp.float32),
        mesh=_mesh,
        compiler_params=pltpu.CompilerParams(needs_layout_passes=False),
        scratch_types=[
            pltpu.VMEM((RPW + 1, D), jnp.float32),
            pltpu.VMEM((NSUB, 128, D), jnp.float32),
            pltpu.VMEM((NSUB, 128), jnp.int32),
            pltpu.VMEM((16,), jnp.int32),
            pltpu.VMEM((BATCH,), jnp.int32),
            pltpu.VMEM_SHARED((NS, 128), jnp.int32),
            pltpu.SMEM((128,), jnp.int32),
            pltpu.SemaphoreType.DMA,
        ],
    )
    return f(hp, bsrc, bdl, bcnt)


def _dense_pool_body(h_ref, w_ref, b_ref, o_ref):
    y = jax.lax.dot_general(
        h_ref[...], w_ref[...], (((1,), (0,)), ((), ())),
        preferred_element_type=jnp.float32)
    o_ref[...] = jnp.maximum(y + b_ref[...][None, :], 0.0)


def _dense_pool(h, w, b):
    return pl.pallas_call(
        _dense_pool_body,
        out_shape=jax.ShapeDtypeStruct((h.shape[0], w.shape[1]), jnp.float32),
    )(h, w, b)


def _dense_out_body(h_ref, agg_ref, ws_ref, wn_ref, b_ref, o_ref, *, act):
    y = jax.lax.dot_general(
        h_ref[...], ws_ref[...], (((1,), (0,)), ((), ())),
        preferred_element_type=jnp.float32)
    y = y + jax.lax.dot_general(
        agg_ref[...], wn_ref[...], (((1,), (0,)), ((), ())),
        preferred_element_type=jnp.float32)
    y = y + b_ref[...][None, :]
    if act == "relu":
        y = jnp.maximum(y, 0.0)
    elif act == "sigmoid":
        y = 1.0 / (1.0 + jnp.exp(-y))
    o_ref[...] = y


def _dense_out(h, agg, ws, wn, b, act):
    return pl.pallas_call(
        functools.partial(_dense_out_body, act=act),
        out_shape=jax.ShapeDtypeStruct((h.shape[0], ws.shape[1]), jnp.float32),
    )(h, agg, ws, wn, b)


def kernel(x, edge_index, Wp0, bp0, Ws0, Wn0, b0, Wp1, bp1, Ws1, Wn1, b1,
           Wp2, bp2, Ws2, Wn2, b2):
    src = edge_index[0]
    dst = edge_index[1]
    bsrc, bdl, bcnt = _bucket(src, dst)
    Ws2p = jnp.pad(Ws2, ((0, 0), (0, D - Ws2.shape[1])))
    Wn2p = jnp.pad(Wn2, ((0, 0), (0, D - Wn2.shape[1])))
    b2p = jnp.pad(b2, (0, D - b2.shape[0]))
    params = [(Wp0, bp0, Ws0, Wn0, b0, "relu"),
              (Wp1, bp1, Ws1, Wn1, b1, "relu"),
              (Wp2, bp2, Ws2p, Wn2p, b2p, "sigmoid")]
    h = x
    for (Wp, bp, Ws, Wn, b, act) in params:
        hp = _dense_pool(h, Wp, bp)
        agg = _segmax(hp, bsrc, bdl, bcnt)
        h = _dense_out(h, agg, Ws, Wn, b, act)
    return h[:, 0]


# speedup vs baseline: 4.1827x; 1.1743x over previous
import functools

import jax
import jax.numpy as jnp
from jax import lax
from jax.experimental import pallas as pl
from jax.experimental.pallas import tpu as pltpu
from jax.experimental.pallas import tpu_sc as plsc

N = 10000
D = 128
E = 320000
NC = 2
NS = 16
NW = NC * NS
RPW = 320
LASTR = N - (NW - 1) * RPW
CHUNK = 2560
NCHUNK = E // CHUNK
BATCH = 512
GUNROLL = 8
NSUB = BATCH // 128
TRASH = RPW
CAP = E + NCHUNK * 16 + CHUNK + 16 + BATCH
CAP = ((CAP + 127) // 128) * 128

_mesh = plsc.VectorSubcoreMesh(core_axis_name="c", subcore_axis_name="s")


def _scalar(x):
    return jnp.max(x) if x.ndim else x


def _wid():
    return lax.axis_index("s") * NC + lax.axis_index("c")


def _bucket_body(src_hbm, dst_hbm, bsrc_hbm, bdl_hbm, bcnt_hbm,
                 src_v, dst_v, sel_src, sel_dl, pad_v, cnt_v):
    wid = _wid()
    lo = wid * RPW
    hi = jnp.minimum(lo + RPW, N)
    lanes = lax.iota(jnp.int32, 16)
    pad_src = wid * 16 + lanes
    trash_v = jnp.full((16,), TRASH, jnp.int32)

    def chunk_body(c, total):
        pltpu.sync_copy(src_hbm.at[pl.ds(c * CHUNK, CHUNK)], src_v)
        pltpu.sync_copy(dst_hbm.at[pl.ds(c * CHUNK, CHUNK)], dst_v)

        def grp(i, nsel_vec):
            gs = [i * GUNROLL + u for u in range(GUNROLL)]
            ds_ = [dst_v[pl.ds(g * 16, 16)] for g in gs]
            ss = [src_v[pl.ds(g * 16, 16)] for g in gs]
            ms = [(d >= lo) & (d < hi) for d in ds_]
            css = [plsc.cumsum(m.astype(jnp.int32)) for m in ms]
            pcs = []
            for m in ms:
                pc = plsc.all_reduce_population_count(m)
                if pc.ndim == 0:
                    pc = jnp.full((16,), pc, jnp.int32)
                pcs.append(pc)
            poss = []
            for u in range(GUNROLL):
                poss.append(nsel_vec + css[u] - 1)
                nsel_vec = nsel_vec + pcs[u]
            for u in range(GUNROLL):
                plsc.store_scatter(sel_src, [poss[u]], ss[u], mask=ms[u])
                plsc.store_scatter(sel_dl, [poss[u]], ds_[u] - lo,
                                   mask=ms[u])
            return nsel_vec

        nsel_vec = lax.fori_loop(0, CHUNK // (16 * GUNROLL), grp,
                                 jnp.zeros((16,), jnp.int32))
        plsc.store_scatter(sel_src, [nsel_vec + lanes], pad_src)
        plsc.store_scatter(sel_dl, [nsel_vec + lanes], trash_v)
        nsel = jnp.bitwise_and(jnp.max(nsel_vec) + 15, -16)
        off = pl.multiple_of(wid * CAP + total, 16)
        pltpu.sync_copy(sel_src, bsrc_hbm.at[pl.ds(off, CHUNK + 16)])
        pltpu.sync_copy(sel_dl, bdl_hbm.at[pl.ds(off, CHUNK + 16)])
        return total + nsel

    total = lax.fori_loop(0, NCHUNK, chunk_body, 0)
    for k in range(BATCH // 16):
        pad_v[pl.ds(k * 16, 16)] = pad_src
    off = pl.multiple_of(wid * CAP + total, 16)
    pltpu.sync_copy(pad_v, bsrc_hbm.at[pl.ds(off, BATCH)])
    for k in range(BATCH // 16):
        pad_v[pl.ds(k * 16, 16)] = trash_v
    pltpu.sync_copy(pad_v, bdl_hbm.at[pl.ds(off, BATCH)])
    cnt_v[...] = jnp.full((16,), total, jnp.int32)
    pltpu.sync_copy(cnt_v, bcnt_hbm.at[pl.ds(pl.multiple_of(wid * 16, 16), 16)])


def _bucket(src, dst):
    f = pl.kernel(
        _bucket_body,
        out_type=(
            jax.ShapeDtypeStruct((NW * CAP,), jnp.int32),
            jax.ShapeDtypeStruct((NW * CAP,), jnp.int32),
            jax.ShapeDtypeStruct((NW * 16,), jnp.int32),
        ),
        mesh=_mesh,
        compiler_params=pltpu.CompilerParams(needs_layout_passes=False),
        scratch_types=[
            pltpu.VMEM((CHUNK,), jnp.int32),
            pltpu.VMEM((CHUNK,), jnp.int32),
            pltpu.VMEM((CHUNK + 16,), jnp.int32),
            pltpu.VMEM((CHUNK + 16,), jnp.int32),
            pltpu.VMEM((BATCH,), jnp.int32),
            pltpu.VMEM((16,), jnp.int32),
        ],
    )
    return f(src, dst)


def _segmax_body(hp_hbm, bsrc_hbm, bdl_hbm, bcnt_hbm, out_hbm,
                 agg_v, rows_v, idx_v, cnt_v, dl_v, shared, dl_smem, sem):
    wid = _wid()
    sid = lax.axis_index("s")
    lo = wid * RPW
    zero = jnp.zeros((16,), jnp.float32)

    def zr(r, _):
        for k in range(D // 16):
            agg_v[r, pl.ds(k * 16, 16)] = zero
        return 0

    lax.fori_loop(0, RPW + 1, zr, 0)

    pltpu.sync_copy(bcnt_hbm.at[pl.ds(pl.multiple_of(wid * 16, 16), 16)], cnt_v)
    cnt = jnp.max(cnt_v[...])
    nb = lax.div(cnt + BATCH - 1, BATCH)

    def batch(b, _):
        boff = pl.multiple_of(wid * CAP + b * BATCH, 16)
        for j in range(NSUB):
            pltpu.sync_copy(bsrc_hbm.at[pl.ds(boff + j * 128, 128)],
                            idx_v.at[j])
        pltpu.sync_copy(bdl_hbm.at[pl.ds(boff, BATCH)], dl_v)

        def issue(j):
            return pltpu.async_copy(hp_hbm.at[idx_v.at[j]], rows_v.at[j], sem)

        def update(j):
            sls = [pl.ds(k * 16, 16) for k in range(D // 16)]

            def upd(e4, _):
                es = [e4 * 4 + u for u in range(4)]
                dds = [dl_smem[e] for e in es]
                rr = [[rows_v[j, e, sl] for sl in sls] for e in es]
                for u in range(4):
                    dd = dds[u]
                    a = [agg_v[dd, sl] for sl in sls]
                    mx = [jnp.maximum(x, y) for x, y in zip(a, rr[u])]
                    for sl, v in zip(sls, mx):
                        agg_v[dd, sl] = v
                return 0
            lax.fori_loop(0, 128 // 4, upd, 0)

        cp = issue(0)
        for j in range(NSUB):
            pltpu.sync_copy(dl_v.at[pl.ds(j * 128, 128)], shared.at[sid])
            pltpu.sync_copy(shared.at[sid], dl_smem)
            cp.wait()
            if j + 1 < NSUB:
                cp = issue(j + 1)
            update(j)
        return 0

    lax.fori_loop(0, nb, batch, 0)

    @pl.when(wid < NW - 1)
    def _():
        pltpu.sync_copy(agg_v.at[pl.ds(0, RPW)], out_hbm.at[pl.ds(lo, RPW)])

    @pl.when(wid == NW - 1)
    def _():
        pltpu.sync_copy(agg_v.at[pl.ds(0, LASTR)], out_hbm.at[pl.ds(lo, LASTR)])


def _segmax(hp, bsrc, bdl, bcnt):
    f = pl.kernel(
        _segmax_body,
        out_type=jax.ShapeDtypeStruct((N, D), jnp.float32),
        mesh=_mesh,
        compiler_params=pltpu.CompilerParams(needs_layout_passes=False),
        scratch_types=[
            pltpu.VMEM((RPW + 1, D), jnp.float32),
            pltpu.VMEM((NSUB, 128, D), jnp.float32),
            pltpu.VMEM((NSUB, 128), jnp.int32),
            pltpu.VMEM((16,), jnp.int32),
            pltpu.VMEM((BATCH,), jnp.int32),
            pltpu.VMEM_SHARED((NS, 128), jnp.int32),
            pltpu.SMEM((128,), jnp.int32),
            pltpu.SemaphoreType.DMA,
        ],
    )
    return f(hp, bsrc, bdl, bcnt)


def _dense_pool_body(h_ref, w_ref, b_ref, o_ref):
    y = jax.lax.dot_general(
        h_ref[...], w_ref[...], (((1,), (0,)), ((), ())),
        preferred_element_type=jnp.float32)
    o_ref[...] = jnp.maximum(y + b_ref[...][None, :], 0.0)


def _dense_pool(h, w, b):
    return pl.pallas_call(
        _dense_pool_body,
        out_shape=jax.ShapeDtypeStruct((h.shape[0], w.shape[1]), jnp.float32),
    )(h, w, b)


def _dense_out_body(h_ref, agg_ref, ws_ref, wn_ref, b_ref, o_ref, *, act):
    y = jax.lax.dot_general(
        h_ref[...], ws_ref[...], (((1,), (0,)), ((), ())),
        preferred_element_type=jnp.float32)
    y = y + jax.lax.dot_general(
        agg_ref[...], wn_ref[...], (((1,), (0,)), ((), ())),
        preferred_element_type=jnp.float32)
    y = y + b_ref[...][None, :]
    if act == "relu":
        y = jnp.maximum(y, 0.0)
    elif act == "sigmoid":
        y = 1.0 / (1.0 + jnp.exp(-y))
    o_ref[...] = y


def _dense_out(h, agg, ws, wn, b, act):
    return pl.pallas_call(
        functools.partial(_dense_out_body, act=act),
        out_shape=jax.ShapeDtypeStruct((h.shape[0], ws.shape[1]), jnp.float32),
    )(h, agg, ws, wn, b)


def kernel(x, edge_index, Wp0, bp0, Ws0, Wn0, b0, Wp1, bp1, Ws1, Wn1, b1,
           Wp2, bp2, Ws2, Wn2, b2):
    src = edge_index[0]
    dst = edge_index[1]
    bsrc, bdl, bcnt = _bucket(src, dst)
    Ws2p = jnp.pad(Ws2, ((0, 0), (0, D - Ws2.shape[1])))
    Wn2p = jnp.pad(Wn2, ((0, 0), (0, D - Wn2.shape[1])))
    b2p = jnp.pad(b2, (0, D - b2.shape[0]))
    params = [(Wp0, bp0, Ws0, Wn0, b0, "relu"),
              (Wp1, bp1, Ws1, Wn1, b1, "relu"),
              (Wp2, bp2, Ws2p, Wn2p, b2p, "sigmoid")]
    h = x
    for (Wp, bp, Ws, Wn, b, act) in params:
        hp = _dense_pool(h, Wp, bp)
        agg = _segmax(hp, bsrc, bdl, bcnt)
        h = _dense_out(h, agg, Ws, Wn, b, act)
    return h[:, 0]
